# Initial kernel scaffold; baseline (speedup 1.0000x reference)
import jax
import jax.numpy as jnp
from jax.experimental import pallas as pl


def kernel(space_features, space_features_local, all_features, num_entries, Wc1, bc1, Wc2, bc2, Wc3, bc3, Wc4, bc4, Wc5, bc5, Wf1, bf1, Wf2, bf2, Wf3, bf3):
    raise NotImplementedError("write your pallas kernel here")


# speedup vs baseline: 9.1906x; 9.1906x over previous
import functools

import jax
import jax.numpy as jnp
from jax import lax
from jax.experimental import pallas as pl
from jax.experimental.pallas import tpu as pltpu

_K = 10


def _knn_body(q_ref, kt_ref, idx_ref, *, n_keys, k):
    q = q_ref[0]
    kt = kt_ref[0]
    bq = q.shape[0]
    sq_q = (q[:, 0:1] * q[:, 0:1] + q[:, 1:2] * q[:, 1:2]) + q[:, 2:3] * q[:, 2:3]
    sq_k = (kt[0:1, :] * kt[0:1, :] + kt[1:2, :] * kt[1:2, :]) + kt[2:3, :] * kt[2:3, :]
    dot = lax.dot_general(q, kt, (((1,), (0,)), ((), ())),
                          preferred_element_type=jnp.float32)
    d = (sq_q - 2.0 * dot) + sq_k
    lane = lax.broadcasted_iota(jnp.int32, (bq, n_keys), 1)
    cols = []
    for _ in range(k):
        m = jnp.min(d, axis=1, keepdims=True)
        cand = jnp.where(d == m, lane, n_keys)
        ik = jnp.min(cand, axis=1, keepdims=True)
        cols.append(ik)
        d = jnp.where(lane == ik, jnp.inf, d)
    idx_ref[0] = jnp.concatenate(cols, axis=1)


def _knn(sg, sgt, bq, k=_K):
    b, n, _ = sg.shape
    return pl.pallas_call(
        functools.partial(_knn_body, n_keys=n, k=k),
        grid=(b, n // bq),
        in_specs=[
            pl.BlockSpec((1, bq, 3), lambda i, j: (i, j, 0)),
            pl.BlockSpec((1, 3, n), lambda i, j: (i, 0, 0)),
        ],
        out_specs=pl.BlockSpec((1, bq, k), lambda i, j: (i, j, 0)),
        out_shape=jax.ShapeDtypeStruct((b, n, k), jnp.int32),
    )(sg, sgt)


def _sconv_body(idx_ref, f_ref, w_ref, b_ref, o_ref, *, n_keys, k, scale, mode):
    idx = idx_ref[0]
    f = f_ref[0]
    if scale != 1.0:
        f = f * scale
    lane = lax.broadcasted_iota(jnp.int32, (idx.shape[0], n_keys), 1)
    dn = (((1,), (0,)), ((), ()))
    gs = []
    for kk in range(k):
        p = (lane == idx[:, kk:kk + 1]).astype(jnp.float32)
        gs.append(lax.dot_general(p, f, dn,
                                  preferred_element_type=jnp.float32))
    if mode == "single":
        g = jnp.concatenate(gs, axis=1)
        out = lax.dot_general(g, w_ref[...], dn,
                              preferred_element_type=jnp.float32)
    elif mode in ("hi", "hst"):
        prec = lax.Precision.HIGH if mode == "hi" else lax.Precision.HIGHEST
        g = jnp.concatenate(gs, axis=1)
        out = lax.dot_general(g, w_ref[...], dn, precision=prec,
                              preferred_element_type=jnp.float32)
    elif mode == "ksum":
        fdim = f.shape[1]
        out = lax.dot_general(gs[0], w_ref[0:fdim], dn,
                              preferred_element_type=jnp.float32)
        for kk in range(1, k):
            out = out + lax.dot_general(gs[kk], w_ref[kk * fdim:(kk + 1) * fdim],
                                        dn, preferred_element_type=jnp.float32)
    elif mode == "half":
        g = jnp.concatenate(gs, axis=1)
        h = g.shape[1] // 2
        out = (lax.dot_general(g[:, :h], w_ref[:h], dn,
                               preferred_element_type=jnp.float32)
               + lax.dot_general(g[:, h:], w_ref[h:], dn,
                                 preferred_element_type=jnp.float32))
    elif mode == "c256":
        g = jnp.concatenate(gs, axis=1)
        out = (lax.dot_general(g[:, :256], w_ref[:256], dn,
                               preferred_element_type=jnp.float32)
               + lax.dot_general(g[:, 256:], w_ref[256:], dn,
                                 preferred_element_type=jnp.float32))
    elif mode == "r44":
        g = jnp.concatenate(gs, axis=1)
        out = (lax.dot_general(g[:, 256:], w_ref[256:], dn,
                               preferred_element_type=jnp.float32)
               + lax.dot_general(g[:, :256], w_ref[:256], dn,
                                 preferred_element_type=jnp.float32))
    elif mode == "fmaj":
        g = jnp.concatenate(gs, axis=1)
        kf = g.shape[1]
        fdim = f.shape[1]
        rj = lax.broadcasted_iota(jnp.int32, (kf, kf), 0)
        cj = lax.broadcasted_iota(jnp.int32, (kf, kf), 1)
        pmat = ((rj % fdim) * k + rj // fdim == cj).astype(jnp.float32)
        gp = lax.dot_general(g, pmat, dn, precision=lax.Precision.HIGHEST,
                             preferred_element_type=jnp.float32)
        pmat_t = ((cj % fdim) * k + cj // fdim == rj).astype(jnp.float32)
        wp = lax.dot_general(pmat_t, w_ref[...], dn,
                             precision=lax.Precision.HIGHEST,
                             preferred_element_type=jnp.float32)
        out = lax.dot_general(gp, wp, dn, preferred_element_type=jnp.float32)
    o_ref[0] = jnp.maximum(out + b_ref[...], 0.0)


def _sconv(idx, feats, w, bias, bq, scale=1.0, k=_K, mode="single"):
    b, n, f = feats.shape
    o = w.shape[1]
    return pl.pallas_call(
        functools.partial(_sconv_body, n_keys=n, k=k, scale=scale, mode=mode),
        grid=(b, n // bq),
        in_specs=[
            pl.BlockSpec((1, bq, k), lambda i, j: (i, j, 0)),
            pl.BlockSpec((1, n, f), lambda i, j: (i, 0, 0)),
            pl.BlockSpec(w.shape, lambda i, j: (0, 0)),
            pl.BlockSpec((1, o), lambda i, j: (0, 0)),
        ],
        out_specs=pl.BlockSpec((1, bq, o), lambda i, j: (i, j, 0)),
        out_shape=jax.ShapeDtypeStruct((b, n, o), jnp.float32),
    )(idx, feats, w, bias.reshape(1, o))


def _pool_body(f_ref, ft_ref, sg_ref, sl_ref, fo_ref, sgo_ref, slo_ref,
               *, n_in, n_out, jc, pc):
    f = f_ref[0]
    sg = sg_ref[0]
    sl = sl_ref[0]
    s_col = jnp.max(f, axis=1, keepdims=True)
    rank_row = jnp.zeros((1, n_in), jnp.float32)
    s_row = jnp.max(ft_ref[0], axis=0, keepdims=True)
    jlane = lax.broadcasted_iota(jnp.int32, (jc, n_in), 1)
    isub = lax.broadcasted_iota(jnp.int32, (jc, n_in), 0)
    for i0 in range(0, n_in, jc):
        sc = s_col[i0:i0 + jc, :]
        iidx = isub + i0
        gt = (sc > s_row)
        eq = jnp.logical_and(sc == s_row, iidx < jlane)
        contrib = jnp.where(gt, 1.0, 0.0) + jnp.where(eq, 1.0, 0.0)
        rank_row = rank_row + jnp.sum(contrib, axis=0, keepdims=True)
    for p0 in range(0, n_out, pc):
        p_iota = (lax.broadcasted_iota(jnp.int32, (pc, n_in), 0) + p0).astype(jnp.float32)
        p2t = (rank_row == p_iota).astype(jnp.float32)
        dn = (((1,), (0,)), ((), ()))
        fo_ref[0, p0:p0 + pc, :] = lax.dot_general(
            p2t, f, dn, precision=lax.Precision.HIGHEST,
            preferred_element_type=jnp.float32)
        sgo_ref[0, p0:p0 + pc, :] = lax.dot_general(
            p2t, sg, dn, precision=lax.Precision.HIGHEST,
            preferred_element_type=jnp.float32)
        slo_ref[0, p0:p0 + pc, :] = lax.dot_general(
            p2t, sl, dn, precision=lax.Precision.HIGHEST,
            preferred_element_type=jnp.float32)


def _pool(feats, sg, sl, n_out, jc=None, pc=500):
    b, n, f = feats.shape
    if jc is None:
        jc = 256 if n % 256 == 0 else 250
    feats_t = jnp.swapaxes(feats, 1, 2)
    return pl.pallas_call(
        functools.partial(_pool_body, n_in=n, n_out=n_out, jc=jc, pc=pc),
        grid=(b,),
        in_specs=[
            pl.BlockSpec((1, n, f), lambda i: (i, 0, 0)),
            pl.BlockSpec((1, f, n), lambda i: (i, 0, 0)),
            pl.BlockSpec((1, n, 3), lambda i: (i, 0, 0)),
            pl.BlockSpec((1, n, 2), lambda i: (i, 0, 0)),
        ],
        out_specs=[
            pl.BlockSpec((1, n_out, f), lambda i: (i, 0, 0)),
            pl.BlockSpec((1, n_out, 3), lambda i: (i, 0, 0)),
            pl.BlockSpec((1, n_out, 2), lambda i: (i, 0, 0)),
        ],
        out_shape=[
            jax.ShapeDtypeStruct((b, n_out, f), jnp.float32),
            jax.ShapeDtypeStruct((b, n_out, 3), jnp.float32),
            jax.ShapeDtypeStruct((b, n_out, 2), jnp.float32),
        ],
    )(feats, feats_t, sg, sl)


def _mlp_body(x_ref, w1_ref, b1_ref, w2_ref, b2_ref, w3_ref, b3_ref,
              o_ref, acc_ref, *, steps):
    s = pl.program_id(0)

    @pl.when(s == 0)
    def _init():
        acc_ref[...] = jnp.zeros_like(acc_ref)

    acc_ref[...] += lax.dot_general(
        x_ref[...], w1_ref[...], (((1,), (0,)), ((), ())),
        preferred_element_type=jnp.float32)

    @pl.when(s == steps - 1)
    def _fin():
        fc1 = jnp.maximum(acc_ref[...] + b1_ref[...], 0.0)
        fc2 = jnp.maximum(
            lax.dot_general(fc1, w2_ref[...], (((1,), (0,)), ((), ())),
                            preferred_element_type=jnp.float32) + b2_ref[...],
            0.0)
        o_ref[...] = lax.dot_general(
            fc2, w3_ref[...], (((1,), (0,)), ((), ())),
            preferred_element_type=jnp.float32) + b3_ref[...]


def _mlp(x, w1, b1, w2, b2, w3, b3, kc=4250):
    rows, kdim = x.shape
    steps = kdim // kc
    return pl.pallas_call(
        functools.partial(_mlp_body, steps=steps),
        grid=(steps,),
        in_specs=[
            pl.BlockSpec((rows, kc), lambda s: (0, s)),
            pl.BlockSpec((kc, w1.shape[1]), lambda s: (s, 0)),
            pl.BlockSpec((1, w1.shape[1]), lambda s: (0, 0)),
            pl.BlockSpec(w2.shape, lambda s: (0, 0)),
            pl.BlockSpec((1, w2.shape[1]), lambda s: (0, 0)),
            pl.BlockSpec(w3.shape, lambda s: (0, 0)),
            pl.BlockSpec((1, w3.shape[1]), lambda s: (0, 0)),
        ],
        out_specs=pl.BlockSpec((rows, w3.shape[1]), lambda s: (0, 0)),
        out_shape=jax.ShapeDtypeStruct((rows, w3.shape[1]), jnp.float32),
        scratch_shapes=[pltpu.VMEM((rows, w1.shape[1]), jnp.float32)],
    )(x, w1, b1.reshape(1, -1), w2, b2.reshape(1, -1), w3, b3.reshape(1, -1))


def kernel(space_features, space_features_local, all_features, num_entries,
           Wc1, bc1, Wc2, bc2, Wc3, bc3, Wc4, bc4, Wc5, bc5,
           Wf1, bf1, Wf2, bf2, Wf3, bf3):
    del num_entries
    sg = space_features
    sl = space_features_local
    sgt = jnp.swapaxes(sg, 1, 2)

    idx1 = _knn(sg, sgt, bq=256)
    a1 = _sconv(idx1, all_features, Wc1, bc1, bq=256, scale=0.001)
    a2 = _sconv(idx1, a1, Wc2, bc2, bq=256)
    a3 = _sconv(idx1, a2, Wc3, bc3, bq=256, mode="c256")

    f4, sg4, sl4 = _pool(a3, sg, sl, 1000)
    idx4 = _knn(sg4, jnp.swapaxes(sg4, 1, 2), bq=1000)
    a4 = _sconv(idx4, f4, Wc4, bc4, bq=1000)

    f5, sg5, sl5 = _pool(a4, sg4, sl4, 500)
    idx5 = _knn(sg5, jnp.swapaxes(sg5, 1, 2), bq=500)
    a5 = _sconv(idx5, f5, Wc5, bc5, bq=500, mode="c256")

    merged = jnp.concatenate([a5, sg5, sl5], axis=2)
    flat = merged.reshape(merged.shape[0], -1)
    kdim = flat.shape[1]
    kpad = 43520
    xpad = jnp.zeros((8, kpad), jnp.float32).at[:flat.shape[0], :kdim].set(flat)
    w1pad = jnp.zeros((kpad, Wf1.shape[1]), jnp.float32).at[:kdim].set(Wf1)
    logits = _mlp(xpad, w1pad, bf1, Wf2, bf2, Wf3, bf3, kc=4352)
    return logits[:flat.shape[0]]


# speedup vs baseline: 9.8639x; 1.0733x over previous
import functools

import jax
import jax.numpy as jnp
from jax import lax
from jax.experimental import pallas as pl
from jax.experimental.pallas import tpu as pltpu

_K = 10


def _knn_body(q_ref, kt_ref, idx_ref, *, n_keys, k):
    q = q_ref[0]
    kt = kt_ref[0]
    bq = q.shape[0]
    sq_q = (q[:, 0:1] * q[:, 0:1] + q[:, 1:2] * q[:, 1:2]) + q[:, 2:3] * q[:, 2:3]
    sq_k = (kt[0:1, :] * kt[0:1, :] + kt[1:2, :] * kt[1:2, :]) + kt[2:3, :] * kt[2:3, :]
    dot = lax.dot_general(q, kt, (((1,), (0,)), ((), ())),
                          preferred_element_type=jnp.float32)
    d = (sq_q - 2.0 * dot) + sq_k
    lane = lax.broadcasted_iota(jnp.int32, (bq, n_keys), 1)
    cols = []
    for _ in range(k):
        m = jnp.min(d, axis=1, keepdims=True)
        cand = jnp.where(d == m, lane, n_keys)
        ik = jnp.min(cand, axis=1, keepdims=True)
        cols.append(ik)
        d = jnp.where(lane == ik, jnp.inf, d)
    idx_ref[0] = jnp.concatenate(cols, axis=1)


def _knn(sg, sgt, bq, k=_K):
    b, n, _ = sg.shape
    return pl.pallas_call(
        functools.partial(_knn_body, n_keys=n, k=k),
        grid=(b, n // bq),
        in_specs=[
            pl.BlockSpec((1, bq, 3), lambda i, j: (i, j, 0)),
            pl.BlockSpec((1, 3, n), lambda i, j: (i, 0, 0)),
        ],
        out_specs=pl.BlockSpec((1, bq, k), lambda i, j: (i, j, 0)),
        out_shape=jax.ShapeDtypeStruct((b, n, k), jnp.int32),
    )(sg, sgt)


def _sconv_body(idx_ref, f_ref, w_ref, b_ref, o_ref, *, n_keys, k, scale, mode):
    idx = idx_ref[0]
    f = f_ref[0]
    if scale != 1.0:
        f = f * scale
    lane = lax.broadcasted_iota(jnp.int32, (idx.shape[0], n_keys), 1)
    dn = (((1,), (0,)), ((), ()))
    gs = []
    for kk in range(k):
        p = (lane == idx[:, kk:kk + 1]).astype(jnp.float32)
        gs.append(lax.dot_general(p, f, dn,
                                  preferred_element_type=jnp.float32))
    if mode == "single":
        g = jnp.concatenate(gs, axis=1)
        out = lax.dot_general(g, w_ref[...], dn,
                              preferred_element_type=jnp.float32)
    elif mode in ("hi", "hst"):
        prec = lax.Precision.HIGH if mode == "hi" else lax.Precision.HIGHEST
        g = jnp.concatenate(gs, axis=1)
        out = lax.dot_general(g, w_ref[...], dn, precision=prec,
                              preferred_element_type=jnp.float32)
    elif mode == "ksum":
        fdim = f.shape[1]
        out = lax.dot_general(gs[0], w_ref[0:fdim], dn,
                              preferred_element_type=jnp.float32)
        for kk in range(1, k):
            out = out + lax.dot_general(gs[kk], w_ref[kk * fdim:(kk + 1) * fdim],
                                        dn, preferred_element_type=jnp.float32)
    elif mode == "half":
        g = jnp.concatenate(gs, axis=1)
        h = g.shape[1] // 2
        out = (lax.dot_general(g[:, :h], w_ref[:h], dn,
                               preferred_element_type=jnp.float32)
               + lax.dot_general(g[:, h:], w_ref[h:], dn,
                                 preferred_element_type=jnp.float32))
    elif mode == "c256":
        g = jnp.concatenate(gs, axis=1)
        out = (lax.dot_general(g[:, :256], w_ref[:256], dn,
                               preferred_element_type=jnp.float32)
               + lax.dot_general(g[:, 256:], w_ref[256:], dn,
                                 preferred_element_type=jnp.float32))
    elif mode == "r44":
        g = jnp.concatenate(gs, axis=1)
        out = (lax.dot_general(g[:, 256:], w_ref[256:], dn,
                               preferred_element_type=jnp.float32)
               + lax.dot_general(g[:, :256], w_ref[:256], dn,
                                 preferred_element_type=jnp.float32))
    elif mode == "fmaj":
        g = jnp.concatenate(gs, axis=1)
        kf = g.shape[1]
        fdim = f.shape[1]
        rj = lax.broadcasted_iota(jnp.int32, (kf, kf), 0)
        cj = lax.broadcasted_iota(jnp.int32, (kf, kf), 1)
        pmat = ((rj % fdim) * k + rj // fdim == cj).astype(jnp.float32)
        gp = lax.dot_general(g, pmat, dn, precision=lax.Precision.HIGHEST,
                             preferred_element_type=jnp.float32)
        pmat_t = ((cj % fdim) * k + cj // fdim == rj).astype(jnp.float32)
        wp = lax.dot_general(pmat_t, w_ref[...], dn,
                             precision=lax.Precision.HIGHEST,
                             preferred_element_type=jnp.float32)
        out = lax.dot_general(gp, wp, dn, preferred_element_type=jnp.float32)
    o_ref[0] = jnp.maximum(out + b_ref[...], 0.0)


def _sconv(idx, feats, w, bias, bq, scale=1.0, k=_K, mode="single"):
    b, n, f = feats.shape
    o = w.shape[1]
    return pl.pallas_call(
        functools.partial(_sconv_body, n_keys=n, k=k, scale=scale, mode=mode),
        grid=(b, n // bq),
        in_specs=[
            pl.BlockSpec((1, bq, k), lambda i, j: (i, j, 0)),
            pl.BlockSpec((1, n, f), lambda i, j: (i, 0, 0)),
            pl.BlockSpec(w.shape, lambda i, j: (0, 0)),
            pl.BlockSpec((1, o), lambda i, j: (0, 0)),
        ],
        out_specs=pl.BlockSpec((1, bq, o), lambda i, j: (i, j, 0)),
        out_shape=jax.ShapeDtypeStruct((b, n, o), jnp.float32),
    )(idx, feats, w, bias.reshape(1, o))


def _pool_body(f_ref, ft_ref, sg_ref, sl_ref, fo_ref, sgo_ref, slo_ref,
               *, n_in, n_out, jc, pc):
    f = f_ref[0]
    sg = sg_ref[0]
    sl = sl_ref[0]
    s_col = jnp.max(f, axis=1, keepdims=True)
    rank_row = jnp.zeros((1, n_in), jnp.float32)
    s_row = jnp.max(ft_ref[0], axis=0, keepdims=True)
    jlane = lax.broadcasted_iota(jnp.int32, (jc, n_in), 1)
    isub = lax.broadcasted_iota(jnp.int32, (jc, n_in), 0)
    for i0 in range(0, n_in, jc):
        sc = s_col[i0:i0 + jc, :]
        iidx = isub + i0
        gt = (sc > s_row)
        eq = jnp.logical_and(sc == s_row, iidx < jlane)
        contrib = jnp.where(gt, 1.0, 0.0) + jnp.where(eq, 1.0, 0.0)
        rank_row = rank_row + jnp.sum(contrib, axis=0, keepdims=True)
    for p0 in range(0, n_out, pc):
        p_iota = (lax.broadcasted_iota(jnp.int32, (pc, n_in), 0) + p0).astype(jnp.float32)
        p2t = (rank_row == p_iota).astype(jnp.float32)
        dn = (((1,), (0,)), ((), ()))
        fo_ref[0, p0:p0 + pc, :] = lax.dot_general(
            p2t, f, dn, precision=lax.Precision.HIGHEST,
            preferred_element_type=jnp.float32)
        sgo_ref[0, p0:p0 + pc, :] = lax.dot_general(
            p2t, sg, dn, precision=lax.Precision.HIGHEST,
            preferred_element_type=jnp.float32)
        slo_ref[0, p0:p0 + pc, :] = lax.dot_general(
            p2t, sl, dn, precision=lax.Precision.HIGHEST,
            preferred_element_type=jnp.float32)


def _pool(feats, sg, sl, n_out, jc=None, pc=500):
    b, n, f = feats.shape
    if jc is None:
        jc = 256 if n % 256 == 0 else 250
    feats_t = jnp.swapaxes(feats, 1, 2)
    return pl.pallas_call(
        functools.partial(_pool_body, n_in=n, n_out=n_out, jc=jc, pc=pc),
        grid=(b,),
        in_specs=[
            pl.BlockSpec((1, n, f), lambda i: (i, 0, 0)),
            pl.BlockSpec((1, f, n), lambda i: (i, 0, 0)),
            pl.BlockSpec((1, n, 3), lambda i: (i, 0, 0)),
            pl.BlockSpec((1, n, 2), lambda i: (i, 0, 0)),
        ],
        out_specs=[
            pl.BlockSpec((1, n_out, f), lambda i: (i, 0, 0)),
            pl.BlockSpec((1, n_out, 3), lambda i: (i, 0, 0)),
            pl.BlockSpec((1, n_out, 2), lambda i: (i, 0, 0)),
        ],
        out_shape=[
            jax.ShapeDtypeStruct((b, n_out, f), jnp.float32),
            jax.ShapeDtypeStruct((b, n_out, 3), jnp.float32),
            jax.ShapeDtypeStruct((b, n_out, 2), jnp.float32),
        ],
    )(feats, feats_t, sg, sl)


def _mlp_body(x_ref, w1_ref, b1_ref, w2_ref, b2_ref, w3_ref, b3_ref,
              o_ref, acc_ref, *, steps):
    s = pl.program_id(0)

    @pl.when(s == 0)
    def _init():
        acc_ref[...] = jnp.zeros_like(acc_ref)

    acc_ref[...] += lax.dot_general(
        x_ref[...], w1_ref[...], (((1,), (0,)), ((), ())),
        preferred_element_type=jnp.float32)

    @pl.when(s == steps - 1)
    def _fin():
        fc1 = jnp.maximum(acc_ref[...] + b1_ref[...], 0.0)
        fc2 = jnp.maximum(
            lax.dot_general(fc1, w2_ref[...], (((1,), (0,)), ((), ())),
                            preferred_element_type=jnp.float32) + b2_ref[...],
            0.0)
        o_ref[...] = lax.dot_general(
            fc2, w3_ref[...], (((1,), (0,)), ((), ())),
            preferred_element_type=jnp.float32) + b3_ref[...]


def _mlp(x, w1, b1, w2, b2, w3, b3, kc=4250):
    rows, kdim = x.shape
    steps = kdim // kc
    return pl.pallas_call(
        functools.partial(_mlp_body, steps=steps),
        grid=(steps,),
        in_specs=[
            pl.BlockSpec((rows, kc), lambda s: (0, s)),
            pl.BlockSpec((kc, w1.shape[1]), lambda s: (s, 0)),
            pl.BlockSpec((1, w1.shape[1]), lambda s: (0, 0)),
            pl.BlockSpec(w2.shape, lambda s: (0, 0)),
            pl.BlockSpec((1, w2.shape[1]), lambda s: (0, 0)),
            pl.BlockSpec(w3.shape, lambda s: (0, 0)),
            pl.BlockSpec((1, w3.shape[1]), lambda s: (0, 0)),
        ],
        out_specs=pl.BlockSpec((rows, w3.shape[1]), lambda s: (0, 0)),
        out_shape=jax.ShapeDtypeStruct((rows, w3.shape[1]), jnp.float32),
        scratch_shapes=[pltpu.VMEM((rows, w1.shape[1]), jnp.float32)],
    )(x, w1, b1.reshape(1, -1), w2, b2.reshape(1, -1), w3, b3.reshape(1, -1))


def kernel(space_features, space_features_local, all_features, num_entries,
           Wc1, bc1, Wc2, bc2, Wc3, bc3, Wc4, bc4, Wc5, bc5,
           Wf1, bf1, Wf2, bf2, Wf3, bf3):
    del num_entries
    sg = space_features
    sl = space_features_local
    sgt = jnp.swapaxes(sg, 1, 2)

    idx1 = _knn(sg, sgt, bq=512)
    a1 = _sconv(idx1, all_features, Wc1, bc1, bq=512, scale=0.001)
    a2 = _sconv(idx1, a1, Wc2, bc2, bq=512)
    a3 = _sconv(idx1, a2, Wc3, bc3, bq=512, mode="c256")

    f4, sg4, sl4 = _pool(a3, sg, sl, 1000)
    idx4 = _knn(sg4, jnp.swapaxes(sg4, 1, 2), bq=1000)
    a4 = _sconv(idx4, f4, Wc4, bc4, bq=1000)

    f5, sg5, sl5 = _pool(a4, sg4, sl4, 500)
    idx5 = _knn(sg5, jnp.swapaxes(sg5, 1, 2), bq=500)
    a5 = _sconv(idx5, f5, Wc5, bc5, bq=500, mode="c256")

    merged = jnp.concatenate([a5, sg5, sl5], axis=2)
    flat = merged.reshape(merged.shape[0], -1)
    kdim = flat.shape[1]
    kpad = 43520
    xpad = jnp.zeros((8, kpad), jnp.float32).at[:flat.shape[0], :kdim].set(flat)
    w1pad = jnp.zeros((kpad, Wf1.shape[1]), jnp.float32).at[:kdim].set(Wf1)
    logits = _mlp(xpad, w1pad, bf1, Wf2, bf2, Wf3, bf3, kc=4352)
    return logits[:flat.shape[0]]
